# Initial kernel scaffold; baseline (speedup 1.0000x reference)
import jax
import jax.numpy as jnp
from jax.experimental import pallas as pl


def kernel(input, target):
    raise NotImplementedError("write your pallas kernel here")


# speedup vs baseline: 12.7461x; 12.7461x over previous
import functools

import jax
import jax.numpy as jnp
from jax import lax
from jax.experimental import pallas as pl
from jax.experimental.pallas import tpu as pltpu
from jax.experimental.pallas import tpu_sc as plsc

N = 16384
D = 128
NC = 2
NS = 16
NW = NC * NS
ROWS_PER_W = N // NW
CHUNK = 128
NCHUNK = ROWS_PER_W // CHUNK


def _shift_right(m, k, fill):
    return jnp.concatenate(
        [jnp.full((1, k), fill, m.dtype), m[:, : m.shape[1] - k]], axis=1)


def _scan_body(t_ref, w_ref, prev_ref):
    t = t_ref[...]
    n = t.shape[1]
    pos = lax.broadcasted_iota(jnp.int32, (1, n), 1)
    m0 = jnp.where(t == 0, pos, -1)
    m1 = jnp.where(t != 0, pos, -1)
    k = 1
    while k < n:
        m0 = jnp.maximum(m0, _shift_right(m0, k, -1))
        m1 = jnp.maximum(m1, _shift_right(m1, k, -1))
        k *= 2
    prev0 = _shift_right(m0, 1, -1)
    prev1 = _shift_right(m1, 1, -1)
    prev = jnp.where(t == 0, prev0, prev1)
    valid = prev >= 0
    count1 = jnp.sum(t)
    count0 = n - count1
    inv0 = 127.0 / jnp.maximum(count0, 1).astype(jnp.float32)
    inv1 = 127.0 / jnp.maximum(count1, 1).astype(jnp.float32)
    w = jnp.where(valid, jnp.where(t == 0, inv0, -inv1), 0.0)
    w_ref[...] = w.astype(jnp.float32)
    prev_ref[...] = jnp.maximum(prev, 0)


def _norm_body(x_ref, w_ref, u_ref, v_ref):
    x = x_ref[...]
    w = w_ref[...]
    c = x - jnp.mean(x, axis=1, keepdims=True)
    q = jnp.sum(c * c, axis=1, keepdims=True)
    u = c * lax.rsqrt(q)
    u_ref[...] = u
    v_ref[...] = u * w


def _final_body(p_ref, o_ref):
    s = jnp.sum(p_ref[...])
    o_ref[...] = jnp.broadcast_to(s * s, (1, 1))


def _pair_body(u_hbm, v_hbm, idx_hbm, out_hbm, idx_v, g_v, vv_v, acc_v, sem):
    wid = lax.axis_index("s") * NC + lax.axis_index("c")
    base = wid * ROWS_PER_W
    pltpu.sync_copy(idx_hbm.at[pl.ds(base, ROWS_PER_W)], idx_v)
    acc = jnp.zeros((16,), jnp.float32)
    for ci in range(NCHUNK):
        r0 = base + ci * CHUNK
        pltpu.async_copy(
            u_hbm.at[idx_v.at[pl.ds(ci * CHUNK, CHUNK)]], g_v, sem).wait()
        pltpu.sync_copy(v_hbm.at[pl.ds(r0, CHUNK)], vv_v)

        def body(r, a):
            for k in range(D // 16):
                a = a + (g_v[r, pl.ds(k * 16, 16)]
                         * vv_v[r, pl.ds(k * 16, 16)])
            return a

        acc = lax.fori_loop(0, CHUNK, body, acc)
    acc_v[...] = acc
    pltpu.sync_copy(acc_v, out_hbm.at[wid])


@functools.cache
def _make_pair_call():
    mesh = plsc.VectorSubcoreMesh(core_axis_name="c", subcore_axis_name="s")
    return functools.partial(
        pl.kernel,
        mesh=mesh,
        out_type=jax.ShapeDtypeStruct((NW, 16), jnp.float32),
        scratch_types=[
            pltpu.VMEM((ROWS_PER_W,), jnp.int32),
            pltpu.VMEM((CHUNK, D), jnp.float32),
            pltpu.VMEM((CHUNK, D), jnp.float32),
            pltpu.VMEM((16,), jnp.float32),
            pltpu.SemaphoreType.DMA,
        ],
    )(_pair_body)


@jax.jit
def kernel(input, target):
    n, d = input.shape
    t2 = target.reshape(1, n).astype(jnp.int32)
    w, prev = pl.pallas_call(
        _scan_body,
        out_shape=(
            jax.ShapeDtypeStruct((1, n), jnp.float32),
            jax.ShapeDtypeStruct((1, n), jnp.int32),
        ),
    )(t2)

    r_blk = 2048
    u, v = pl.pallas_call(
        _norm_body,
        grid=(n // r_blk,),
        in_specs=[
            pl.BlockSpec((r_blk, d), lambda i: (i, 0)),
            pl.BlockSpec((r_blk, 1), lambda i: (i, 0)),
        ],
        out_specs=[
            pl.BlockSpec((r_blk, d), lambda i: (i, 0)),
            pl.BlockSpec((r_blk, d), lambda i: (i, 0)),
        ],
        out_shape=(
            jax.ShapeDtypeStruct((n, d), jnp.float32),
            jax.ShapeDtypeStruct((n, d), jnp.float32),
        ),
        compiler_params=pltpu.CompilerParams(
            dimension_semantics=("parallel",)),
    )(input, w.reshape(n, 1))

    partials = _make_pair_call()(u, v, prev.reshape(n))

    out = pl.pallas_call(
        _final_body,
        out_shape=jax.ShapeDtypeStruct((1, 1), jnp.float32),
    )(partials)
    return out.reshape(())


# speedup vs baseline: 13.4858x; 1.0580x over previous
import functools

import jax
import jax.numpy as jnp
from jax import lax
from jax.experimental import pallas as pl
from jax.experimental.pallas import tpu as pltpu
from jax.experimental.pallas import tpu_sc as plsc

N = 16384
D = 128
SQ = 128
NC = 2
NS = 16
NW = NC * NS
ROWS_PER_W = N // NW
CHUNK = 128
NCHUNK = ROWS_PER_W // CHUNK


def _shift_lane(m, k, fill):
    r, c = m.shape
    return jnp.concatenate(
        [jnp.full((r, k), fill, m.dtype), m[:, : c - k]], axis=1)


def _shift_sub(m, k, fill):
    r, c = m.shape
    return jnp.concatenate(
        [jnp.full((k, c), fill, m.dtype), m[: r - k, :]], axis=0)


def _scan_body(t_ref, prev_ref, cls_ref, inv0_ref, inv1_ref):
    t = t_ref[...]
    pos = (lax.broadcasted_iota(jnp.int32, (SQ, SQ), 0) * SQ
           + lax.broadcasted_iota(jnp.int32, (SQ, SQ), 1))
    m0 = jnp.where(t == 0, pos, -1)
    m1 = jnp.where(t != 0, pos, -1)
    k = 1
    while k < SQ:
        m0 = jnp.maximum(m0, _shift_lane(m0, k, -1))
        m1 = jnp.maximum(m1, _shift_lane(m1, k, -1))
        k *= 2
    e0 = _shift_sub(m0[:, SQ - 1 : SQ], 1, -1)
    e1 = _shift_sub(m1[:, SQ - 1 : SQ], 1, -1)
    k = 1
    while k < SQ:
        e0 = jnp.maximum(e0, _shift_sub(e0, k, -1))
        e1 = jnp.maximum(e1, _shift_sub(e1, k, -1))
        k *= 2
    prev0 = jnp.maximum(_shift_lane(m0, 1, -1), e0)
    prev1 = jnp.maximum(_shift_lane(m1, 1, -1), e1)
    prev = jnp.where(t == 0, prev0, prev1)
    valid = prev >= 0
    cls = jnp.where(valid, t, 2)
    count1 = jnp.sum(t)
    count0 = SQ * SQ - count1
    inv0 = 127.0 / jnp.maximum(count0, 1).astype(jnp.float32)
    inv1 = 127.0 / jnp.maximum(count1, 1).astype(jnp.float32)
    prev_ref[...] = jnp.maximum(prev, 0)
    cls_ref[...] = cls
    inv0_ref[...] = jnp.broadcast_to(inv0, (1, 1))
    inv1_ref[...] = jnp.broadcast_to(inv1, (1, 1))


def _norm_body(x_ref, u_ref):
    x = x_ref[...]
    c = x - jnp.mean(x, axis=1, keepdims=True)
    q = jnp.sum(c * c, axis=1, keepdims=True)
    u_ref[...] = c * lax.rsqrt(q)


def _final_body(p_ref, inv0_ref, inv1_ref, o_ref):
    p = p_ref[...]
    s0 = jnp.broadcast_to(jnp.sum(p[:, :16]), (1, 1))
    s1 = jnp.broadcast_to(jnp.sum(p[:, 16:]), (1, 1))
    d = inv0_ref[...] * s0 - inv1_ref[...] * s1
    o_ref[...] = d * d


def _pair_body(u_hbm, idx_hbm, cls_hbm, out_hbm,
               idx_v, cls_v, g0, g1, l0, l1, acc_v,
               sg0, sg1, sl0, sl1):
    wid = lax.axis_index("s") * NC + lax.axis_index("c")
    base = wid * ROWS_PER_W
    pltpu.sync_copy(idx_hbm.at[pl.ds(base, ROWS_PER_W)], idx_v)
    pltpu.sync_copy(cls_hbm.at[pl.ds(base, ROWS_PER_W)], cls_v)

    gb = (g0, g1)
    lb = (l0, l1)
    sg = (sg0, sg1)
    sl = (sl0, sl1)

    def start(ci):
        s = ci % 2
        cg = pltpu.async_copy(
            u_hbm.at[idx_v.at[pl.ds(ci * CHUNK, CHUNK)]], gb[s], sg[s])
        cl = pltpu.async_copy(
            u_hbm.at[pl.ds(base + ci * CHUNK, CHUNK)], lb[s], sl[s])
        return cg, cl

    acc0 = jnp.zeros((16,), jnp.float32)
    acc1 = jnp.zeros((16,), jnp.float32)
    pend = start(0)
    for ci in range(NCHUNK):
        s = ci % 2
        cur = pend
        if ci + 1 < NCHUNK:
            pend = start(ci + 1)
        cur[0].wait()
        cur[1].wait()
        gv, lv = gb[s], lb[s]

        def body(i, a, ci=ci, gv=gv, lv=lv):
            a0, a1 = a
            cv = cls_v[pl.ds(ci * CHUNK + i * 16, 16)]
            for j in range(16):
                r = i * 16 + j
                rp = gv[r, pl.ds(0, 16)] * lv[r, pl.ds(0, 16)]
                for k in range(1, D // 16):
                    rp = rp + (gv[r, pl.ds(k * 16, 16)]
                               * lv[r, pl.ds(k * 16, 16)])
                c = cv[j]
                a0 = a0 + jnp.where(c == 0, rp, 0.0)
                a1 = a1 + jnp.where(c == 1, rp, 0.0)
            return a0, a1

        acc0, acc1 = lax.fori_loop(0, CHUNK // 16, body, (acc0, acc1))
    acc_v[pl.ds(0, 16)] = acc0
    acc_v[pl.ds(16, 16)] = acc1
    pltpu.sync_copy(acc_v, out_hbm.at[wid])


@functools.cache
def _make_pair_call():
    mesh = plsc.VectorSubcoreMesh(core_axis_name="c", subcore_axis_name="s")
    return functools.partial(
        pl.kernel,
        mesh=mesh,
        out_type=jax.ShapeDtypeStruct((NW, 32), jnp.float32),
        scratch_types=[
            pltpu.VMEM((ROWS_PER_W,), jnp.int32),
            pltpu.VMEM((ROWS_PER_W,), jnp.int32),
            pltpu.VMEM((CHUNK, D), jnp.float32),
            pltpu.VMEM((CHUNK, D), jnp.float32),
            pltpu.VMEM((CHUNK, D), jnp.float32),
            pltpu.VMEM((CHUNK, D), jnp.float32),
            pltpu.VMEM((32,), jnp.float32),
            pltpu.SemaphoreType.DMA,
            pltpu.SemaphoreType.DMA,
            pltpu.SemaphoreType.DMA,
            pltpu.SemaphoreType.DMA,
        ],
    )(_pair_body)


@jax.jit
def kernel(input, target):
    n, d = input.shape
    t2 = target.reshape(SQ, SQ).astype(jnp.int32)
    prev, cls, inv0, inv1 = pl.pallas_call(
        _scan_body,
        out_shape=(
            jax.ShapeDtypeStruct((SQ, SQ), jnp.int32),
            jax.ShapeDtypeStruct((SQ, SQ), jnp.int32),
            jax.ShapeDtypeStruct((1, 1), jnp.float32),
            jax.ShapeDtypeStruct((1, 1), jnp.float32),
        ),
    )(t2)

    r_blk = 2048
    u = pl.pallas_call(
        _norm_body,
        grid=(n // r_blk,),
        in_specs=[pl.BlockSpec((r_blk, d), lambda i: (i, 0))],
        out_specs=pl.BlockSpec((r_blk, d), lambda i: (i, 0)),
        out_shape=jax.ShapeDtypeStruct((n, d), jnp.float32),
        compiler_params=pltpu.CompilerParams(
            dimension_semantics=("parallel",)),
    )(input)

    partials = _make_pair_call()(u, prev.reshape(n), cls.reshape(n))

    out = pl.pallas_call(
        _final_body,
        out_shape=jax.ShapeDtypeStruct((1, 1), jnp.float32),
    )(partials, inv0, inv1)
    return out.reshape(())


# speedup vs baseline: 14.8472x; 1.1009x over previous
import functools

import jax
import jax.numpy as jnp
from jax import lax
from jax.experimental import pallas as pl
from jax.experimental.pallas import tpu as pltpu
from jax.experimental.pallas import tpu_sc as plsc

N = 16384
D = 128
SQ = 128
NC = 2
NS = 16
NW = NC * NS
ROWS_PER_W = N // NW
CHUNK = 128
NCHUNK = ROWS_PER_W // CHUNK


def _shift_lane(m, k, fill):
    r, c = m.shape
    return jnp.concatenate(
        [jnp.full((r, k), fill, m.dtype), m[:, : c - k]], axis=1)


def _shift_sub(m, k, fill):
    r, c = m.shape
    return jnp.concatenate(
        [jnp.full((k, c), fill, m.dtype), m[: r - k, :]], axis=0)


def _scan_body(t_ref, prev_ref, w_ref):
    t = t_ref[...]
    pos = (lax.broadcasted_iota(jnp.int32, (SQ, SQ), 0) * SQ
           + lax.broadcasted_iota(jnp.int32, (SQ, SQ), 1))
    m0 = jnp.where(t == 0, pos, -1)
    m1 = jnp.where(t != 0, pos, -1)
    k = 1
    while k < SQ:
        m0 = jnp.maximum(m0, _shift_lane(m0, k, -1))
        m1 = jnp.maximum(m1, _shift_lane(m1, k, -1))
        k *= 2
    e0 = _shift_sub(m0[:, SQ - 1 : SQ], 1, -1)
    e1 = _shift_sub(m1[:, SQ - 1 : SQ], 1, -1)
    k = 1
    while k < SQ:
        e0 = jnp.maximum(e0, _shift_sub(e0, k, -1))
        e1 = jnp.maximum(e1, _shift_sub(e1, k, -1))
        k *= 2
    prev0 = jnp.maximum(_shift_lane(m0, 1, -1), e0)
    prev1 = jnp.maximum(_shift_lane(m1, 1, -1), e1)
    prev = jnp.where(t == 0, prev0, prev1)
    valid = prev >= 0
    count1 = jnp.sum(t)
    count0 = SQ * SQ - count1
    inv0 = 127.0 / jnp.maximum(count0, 1).astype(jnp.float32)
    inv1 = 127.0 / jnp.maximum(count1, 1).astype(jnp.float32)
    w = jnp.where(valid, jnp.where(t == 0, inv0, -inv1), 0.0)
    prev_ref[...] = jnp.maximum(prev, 0)
    w_ref[...] = w.astype(jnp.float32)


def _norm_body(x_ref, w_ref, u_ref, v_ref):
    x = x_ref[...]
    w = w_ref[...]
    c = x - jnp.mean(x, axis=1, keepdims=True)
    q = jnp.sum(c * c, axis=1, keepdims=True)
    u = c * lax.rsqrt(q)
    u_ref[...] = u
    v_ref[...] = u * w


def _final_body(p_ref, o_ref):
    s = jnp.sum(p_ref[...])
    o_ref[...] = jnp.broadcast_to(s * s, (1, 1))


def _pair_body(u_hbm, v_hbm, idx_hbm, out_hbm,
               idx_v, g0, g1, l0, l1, acc_v, sg0, sg1, sl0, sl1):
    wid = lax.axis_index("s") * NC + lax.axis_index("c")
    base = wid * ROWS_PER_W
    pltpu.sync_copy(idx_hbm.at[pl.ds(base, ROWS_PER_W)], idx_v)

    gb = (g0, g1)
    lb = (l0, l1)
    sg = (sg0, sg1)
    sl = (sl0, sl1)

    def start(ci):
        s = ci % 2
        cg = pltpu.async_copy(
            u_hbm.at[idx_v.at[pl.ds(ci * CHUNK, CHUNK)]], gb[s], sg[s])
        cl = pltpu.async_copy(
            v_hbm.at[pl.ds(base + ci * CHUNK, CHUNK)], lb[s], sl[s])
        return cg, cl

    acc = jnp.zeros((16,), jnp.float32)
    pend = start(0)
    for ci in range(NCHUNK):
        s = ci % 2
        cur = pend
        if ci + 1 < NCHUNK:
            pend = start(ci + 1)
        cur[0].wait()
        cur[1].wait()
        gv, lv = gb[s], lb[s]

        def body(r, a, gv=gv, lv=lv):
            for k in range(D // 16):
                a = a + (gv[r, pl.ds(k * 16, 16)]
                         * lv[r, pl.ds(k * 16, 16)])
            return a

        acc = lax.fori_loop(0, CHUNK, body, acc)
    acc_v[...] = acc
    pltpu.sync_copy(acc_v, out_hbm.at[wid])


@functools.cache
def _make_pair_call():
    mesh = plsc.VectorSubcoreMesh(core_axis_name="c", subcore_axis_name="s")
    return functools.partial(
        pl.kernel,
        mesh=mesh,
        out_type=jax.ShapeDtypeStruct((NW, 16), jnp.float32),
        scratch_types=[
            pltpu.VMEM((ROWS_PER_W,), jnp.int32),
            pltpu.VMEM((CHUNK, D), jnp.float32),
            pltpu.VMEM((CHUNK, D), jnp.float32),
            pltpu.VMEM((CHUNK, D), jnp.float32),
            pltpu.VMEM((CHUNK, D), jnp.float32),
            pltpu.VMEM((16,), jnp.float32),
            pltpu.SemaphoreType.DMA,
            pltpu.SemaphoreType.DMA,
            pltpu.SemaphoreType.DMA,
            pltpu.SemaphoreType.DMA,
        ],
    )(_pair_body)


@jax.jit
def kernel(input, target):
    n, d = input.shape
    t2 = target.reshape(SQ, SQ).astype(jnp.int32)
    prev, w = pl.pallas_call(
        _scan_body,
        out_shape=(
            jax.ShapeDtypeStruct((SQ, SQ), jnp.int32),
            jax.ShapeDtypeStruct((SQ, SQ), jnp.float32),
        ),
    )(t2)

    r_blk = 2048
    u, v = pl.pallas_call(
        _norm_body,
        grid=(n // r_blk,),
        in_specs=[
            pl.BlockSpec((r_blk, d), lambda i: (i, 0)),
            pl.BlockSpec((r_blk, 1), lambda i: (i, 0)),
        ],
        out_specs=[
            pl.BlockSpec((r_blk, d), lambda i: (i, 0)),
            pl.BlockSpec((r_blk, d), lambda i: (i, 0)),
        ],
        out_shape=(
            jax.ShapeDtypeStruct((n, d), jnp.float32),
            jax.ShapeDtypeStruct((n, d), jnp.float32),
        ),
        compiler_params=pltpu.CompilerParams(
            dimension_semantics=("parallel",)),
    )(input, w.reshape(n, 1))

    partials = _make_pair_call()(u, v, prev.reshape(n))

    out = pl.pallas_call(
        _final_body,
        out_shape=jax.ShapeDtypeStruct((1, 1), jnp.float32),
    )(partials)
    return out.reshape(())


# speedup vs baseline: 17.3432x; 1.1681x over previous
import functools

import jax
import jax.numpy as jnp
from jax import lax
from jax.experimental import pallas as pl
from jax.experimental.pallas import tpu as pltpu
from jax.experimental.pallas import tpu_sc as plsc

N = 16384
D = 128
SQ = 128
NC = 2
NS = 16
NW = NC * NS
ROWS_PER_W = N // NW
CHUNK = 128
NCHUNK = ROWS_PER_W // CHUNK


def _shift_lane(m, k, fill):
    r, c = m.shape
    return jnp.concatenate(
        [jnp.full((r, k), fill, m.dtype), m[:, : c - k]], axis=1)


def _shift_sub(m, k, fill):
    r, c = m.shape
    return jnp.concatenate(
        [jnp.full((k, c), fill, m.dtype), m[: r - k, :]], axis=0)


def _scan_body(t_ref, prev_ref, w_ref):
    t = t_ref[...]
    pos = (lax.broadcasted_iota(jnp.int32, (SQ, SQ), 0) * SQ
           + lax.broadcasted_iota(jnp.int32, (SQ, SQ), 1))
    m0 = jnp.where(t == 0, pos, -1)
    m1 = jnp.where(t != 0, pos, -1)
    k = 1
    while k < SQ:
        m0 = jnp.maximum(m0, _shift_lane(m0, k, -1))
        m1 = jnp.maximum(m1, _shift_lane(m1, k, -1))
        k *= 2
    e0 = _shift_sub(m0[:, SQ - 1 : SQ], 1, -1)
    e1 = _shift_sub(m1[:, SQ - 1 : SQ], 1, -1)
    k = 1
    while k < SQ:
        e0 = jnp.maximum(e0, _shift_sub(e0, k, -1))
        e1 = jnp.maximum(e1, _shift_sub(e1, k, -1))
        k *= 2
    prev0 = jnp.maximum(_shift_lane(m0, 1, -1), e0)
    prev1 = jnp.maximum(_shift_lane(m1, 1, -1), e1)
    prev = jnp.where(t == 0, prev0, prev1)
    valid = prev >= 0
    count1 = jnp.sum(t)
    count0 = SQ * SQ - count1
    inv0 = 127.0 / jnp.maximum(count0, 1).astype(jnp.float32)
    inv1 = 127.0 / jnp.maximum(count1, 1).astype(jnp.float32)
    w = jnp.where(valid, jnp.where(t == 0, inv0, -inv1), 0.0)
    prev_ref[...] = jnp.maximum(prev, 0)
    w_ref[...] = w.astype(jnp.float32)


def _norm_body(x_ref, w_ref, u_ref, v_ref):
    x = x_ref[...]
    w = w_ref[...]
    c = x - jnp.mean(x, axis=2, keepdims=True)
    q = jnp.sum(c * c, axis=2, keepdims=True)
    u = c * lax.rsqrt(q)
    u_ref[...] = u
    v_ref[...] = u * w[:, :, None]


def _final_body(p_ref, o_ref):
    s = jnp.sum(p_ref[...])
    o_ref[...] = jnp.broadcast_to(s * s, (1, 1))


def _pair_body(u_hbm, v_hbm, idx_hbm, out_hbm,
               idx_v, g0, g1, l0, l1, acc_v, sg0, sg1, sl0, sl1):
    wid = lax.axis_index("s") * NC + lax.axis_index("c")
    base = wid * ROWS_PER_W
    pltpu.sync_copy(idx_hbm.at[pl.ds(wid * NCHUNK, NCHUNK)], idx_v)

    gb = (g0, g1)
    lb = (l0, l1)
    sg = (sg0, sg1)
    sl = (sl0, sl1)

    def start(ci):
        s = ci % 2
        cg = pltpu.async_copy(
            u_hbm.at[idx_v.at[ci]], gb[s], sg[s])
        cl = pltpu.async_copy(
            v_hbm.at[pl.ds(base + ci * CHUNK, CHUNK)], lb[s], sl[s])
        return cg, cl

    acc = jnp.zeros((16,), jnp.float32)
    pend = start(0)
    for ci in range(NCHUNK):
        s = ci % 2
        cur = pend
        if ci + 1 < NCHUNK:
            pend = start(ci + 1)
        cur[0].wait()
        cur[1].wait()
        gv, lv = gb[s], lb[s]

        def body(r, a, gv=gv, lv=lv):
            for k in range(D // 16):
                a = a + (gv[r, pl.ds(k * 16, 16)]
                         * lv[r, pl.ds(k * 16, 16)])
            return a

        acc = lax.fori_loop(0, CHUNK, body, acc)
    acc_v[...] = acc
    pltpu.sync_copy(acc_v, out_hbm.at[wid])


@functools.cache
def _make_pair_call():
    mesh = plsc.VectorSubcoreMesh(core_axis_name="c", subcore_axis_name="s")
    return functools.partial(
        pl.kernel,
        mesh=mesh,
        out_type=jax.ShapeDtypeStruct((NW, 16), jnp.float32),
        scratch_types=[
            pltpu.VMEM((NCHUNK, CHUNK), jnp.int32),
            pltpu.VMEM((CHUNK, D), jnp.float32),
            pltpu.VMEM((CHUNK, D), jnp.float32),
            pltpu.VMEM((CHUNK, D), jnp.float32),
            pltpu.VMEM((CHUNK, D), jnp.float32),
            pltpu.VMEM((16,), jnp.float32),
            pltpu.SemaphoreType.DMA,
            pltpu.SemaphoreType.DMA,
            pltpu.SemaphoreType.DMA,
            pltpu.SemaphoreType.DMA,
        ],
    )(_pair_body)


@jax.jit
def kernel(input, target):
    n, d = input.shape
    t2 = target.reshape(SQ, SQ).astype(jnp.int32)
    prev, w = pl.pallas_call(
        _scan_body,
        out_shape=(
            jax.ShapeDtypeStruct((SQ, SQ), jnp.int32),
            jax.ShapeDtypeStruct((SQ, SQ), jnp.float32),
        ),
    )(t2)

    g_blk = 16
    x3 = input.reshape(SQ, SQ, d)
    u3, v3 = pl.pallas_call(
        _norm_body,
        grid=(SQ // g_blk,),
        in_specs=[
            pl.BlockSpec((g_blk, SQ, d), lambda i: (i, 0, 0)),
            pl.BlockSpec((g_blk, SQ), lambda i: (i, 0)),
        ],
        out_specs=[
            pl.BlockSpec((g_blk, SQ, d), lambda i: (i, 0, 0)),
            pl.BlockSpec((g_blk, SQ, d), lambda i: (i, 0, 0)),
        ],
        out_shape=(
            jax.ShapeDtypeStruct((SQ, SQ, d), jnp.float32),
            jax.ShapeDtypeStruct((SQ, SQ, d), jnp.float32),
        ),
        compiler_params=pltpu.CompilerParams(
            dimension_semantics=("parallel",)),
    )(x3, w)

    partials = _make_pair_call()(
        u3.reshape(n, d), v3.reshape(n, d), prev)

    out = pl.pallas_call(
        _final_body,
        out_shape=jax.ShapeDtypeStruct((1, 1), jnp.float32),
    )(partials)
    return out.reshape(())


# speedup vs baseline: 17.9472x; 1.0348x over previous
import functools

import jax
import jax.numpy as jnp
from jax import lax
from jax.experimental import pallas as pl
from jax.experimental.pallas import tpu as pltpu
from jax.experimental.pallas import tpu_sc as plsc

N = 16384
D = 128
SQ = 128
NC = 2
NS = 16
NW = NC * NS
ROWS_PER_W = N // NW
CHUNK = 128
NCHUNK = ROWS_PER_W // CHUNK


def _shift_lane(m, k, fill):
    r, c = m.shape
    return jnp.concatenate(
        [jnp.full((r, k), fill, m.dtype), m[:, : c - k]], axis=1)


def _shift_sub(m, k, fill):
    r, c = m.shape
    return jnp.concatenate(
        [jnp.full((k, c), fill, m.dtype), m[: r - k, :]], axis=0)


def _scan_math(t):
    pos = (lax.broadcasted_iota(jnp.int32, (SQ, SQ), 0) * SQ
           + lax.broadcasted_iota(jnp.int32, (SQ, SQ), 1))
    m0 = jnp.where(t == 0, pos, -1)
    m1 = jnp.where(t != 0, pos, -1)
    k = 1
    while k < SQ:
        m0 = jnp.maximum(m0, _shift_lane(m0, k, -1))
        m1 = jnp.maximum(m1, _shift_lane(m1, k, -1))
        k *= 2
    e0 = _shift_sub(m0[:, SQ - 1 : SQ], 1, -1)
    e1 = _shift_sub(m1[:, SQ - 1 : SQ], 1, -1)
    k = 1
    while k < SQ:
        e0 = jnp.maximum(e0, _shift_sub(e0, k, -1))
        e1 = jnp.maximum(e1, _shift_sub(e1, k, -1))
        k *= 2
    prev0 = jnp.maximum(_shift_lane(m0, 1, -1), e0)
    prev1 = jnp.maximum(_shift_lane(m1, 1, -1), e1)
    prev = jnp.where(t == 0, prev0, prev1)
    valid = prev >= 0
    count1 = jnp.sum(t)
    count0 = SQ * SQ - count1
    inv0 = 127.0 / jnp.maximum(count0, 1).astype(jnp.float32)
    inv1 = 127.0 / jnp.maximum(count1, 1).astype(jnp.float32)
    w = jnp.where(valid, jnp.where(t == 0, inv0, -inv1), 0.0)
    return jnp.maximum(prev, 0), w.astype(jnp.float32)


G_BLK = 16


def _scan_norm_body(t_ref, x_ref, prev_ref, u_ref, v_ref, w_scr):
    step = pl.program_id(0)

    @pl.when(step == 0)
    def _scan():
        prev, w = _scan_math(t_ref[...])
        prev_ref[...] = prev
        w_scr[...] = w

    @pl.when(step > 0)
    def _norm():
        x = x_ref[...]
        g0 = (step - 1) * G_BLK
        w = w_scr[pl.ds(g0, G_BLK), :]
        c = x - jnp.mean(x, axis=2, keepdims=True)
        q = jnp.sum(c * c, axis=2, keepdims=True)
        u = c * lax.rsqrt(q)
        u_ref[...] = u
        v_ref[...] = u * w[:, :, None]


def _final_body(p_ref, o_ref):
    s = jnp.sum(p_ref[...])
    o_ref[...] = jnp.broadcast_to(s * s, (1, 1))


def _pair_body(u_hbm, v_hbm, idx_hbm, out_hbm,
               idx_v, g0, g1, l0, l1, acc_v, sg0, sg1, sl0, sl1):
    wid = lax.axis_index("s") * NC + lax.axis_index("c")
    base = wid * ROWS_PER_W
    pltpu.sync_copy(idx_hbm.at[pl.ds(wid * NCHUNK, NCHUNK)], idx_v)

    gb = (g0, g1)
    lb = (l0, l1)
    sg = (sg0, sg1)
    sl = (sl0, sl1)

    def start(ci):
        s = ci % 2
        cg = pltpu.async_copy(
            u_hbm.at[idx_v.at[ci]], gb[s], sg[s])
        cl = pltpu.async_copy(
            v_hbm.at[pl.ds(base + ci * CHUNK, CHUNK)], lb[s], sl[s])
        return cg, cl

    acc = jnp.zeros((16,), jnp.float32)
    pend = start(0)
    for ci in range(NCHUNK):
        s = ci % 2
        cur = pend
        if ci + 1 < NCHUNK:
            pend = start(ci + 1)
        cur[0].wait()
        cur[1].wait()
        gv, lv = gb[s], lb[s]

        def body(r, a, gv=gv, lv=lv):
            for k in range(D // 16):
                a = a + (gv[r, pl.ds(k * 16, 16)]
                         * lv[r, pl.ds(k * 16, 16)])
            return a

        acc = lax.fori_loop(0, CHUNK, body, acc)
    acc_v[...] = acc
    pltpu.sync_copy(acc_v, out_hbm.at[wid])


@functools.cache
def _make_pair_call():
    mesh = plsc.VectorSubcoreMesh(core_axis_name="c", subcore_axis_name="s")
    return functools.partial(
        pl.kernel,
        mesh=mesh,
        out_type=jax.ShapeDtypeStruct((NW, 16), jnp.float32),
        scratch_types=[
            pltpu.VMEM((NCHUNK, CHUNK), jnp.int32),
            pltpu.VMEM((CHUNK, D), jnp.float32),
            pltpu.VMEM((CHUNK, D), jnp.float32),
            pltpu.VMEM((CHUNK, D), jnp.float32),
            pltpu.VMEM((CHUNK, D), jnp.float32),
            pltpu.VMEM((16,), jnp.float32),
            pltpu.SemaphoreType.DMA,
            pltpu.SemaphoreType.DMA,
            pltpu.SemaphoreType.DMA,
            pltpu.SemaphoreType.DMA,
        ],
    )(_pair_body)


@jax.jit
def kernel(input, target):
    n, d = input.shape
    t2 = target.reshape(SQ, SQ).astype(jnp.int32)
    x3 = input.reshape(SQ, SQ, d)
    prev, u3, v3 = pl.pallas_call(
        _scan_norm_body,
        grid=(SQ // G_BLK + 1,),
        in_specs=[
            pl.BlockSpec((SQ, SQ), lambda i: (0, 0)),
            pl.BlockSpec((G_BLK, SQ, d),
                         lambda i: (jnp.maximum(i - 1, 0), 0, 0)),
        ],
        out_specs=[
            pl.BlockSpec((SQ, SQ), lambda i: (0, 0)),
            pl.BlockSpec((G_BLK, SQ, d),
                         lambda i: (jnp.maximum(i - 1, 0), 0, 0)),
            pl.BlockSpec((G_BLK, SQ, d),
                         lambda i: (jnp.maximum(i - 1, 0), 0, 0)),
        ],
        out_shape=(
            jax.ShapeDtypeStruct((SQ, SQ), jnp.int32),
            jax.ShapeDtypeStruct((SQ, SQ, d), jnp.float32),
            jax.ShapeDtypeStruct((SQ, SQ, d), jnp.float32),
        ),
        scratch_shapes=[pltpu.VMEM((SQ, SQ), jnp.float32)],
    )(t2, x3)

    partials = _make_pair_call()(
        u3.reshape(n, d), v3.reshape(n, d), prev)

    out = pl.pallas_call(
        _final_body,
        out_shape=jax.ShapeDtypeStruct((1, 1), jnp.float32),
    )(partials)
    return out.reshape(())
